# baseline (speedup 1.0000x reference)
import functools

import jax
import jax.numpy as jnp
from jax.experimental import pallas as pl

QBLK = 512


def _threenn_block(u_ref, kt_ref, dist_ref, idx_ref):
    u = u_ref[0]
    kt = kt_ref[0]
    ux, uy, uz = u[:, 0:1], u[:, 1:2], u[:, 2:3]
    kx, ky, kz = kt[0:1, :], kt[1:2, :], kt[2:3, :]

    bf = lambda x: x.astype(jnp.bfloat16).astype(jnp.float32)
    dot = bf(ux) * bf(kx) + bf(uy) * bf(ky) + bf(uz) * bf(kz)
    su = ux * ux + uy * uy + uz * uz
    sk = kx * kx + ky * ky + kz * kz
    d = -2.0 * dot + su + sk

    iota = jax.lax.broadcasted_iota(jnp.int32, d.shape, 1)
    for k in range(3):
        mn = jnp.min(d, axis=1, keepdims=True)
        im = jnp.argmin(d, axis=1, keepdims=True).astype(jnp.int32)
        dist_ref[0, :, pl.ds(k, 1)] = mn
        idx_ref[0, :, pl.ds(k, 1)] = im
        if k < 2:
            d = jnp.where(iota == im, jnp.inf, d)


@jax.jit
def kernel(unknown, known):
    b, n, _ = unknown.shape
    m = known.shape[1]
    kt = known.transpose(0, 2, 1)
    grid = (b, n // QBLK)
    dist, idx = pl.pallas_call(
        _threenn_block,
        grid=grid,
        in_specs=[
            pl.BlockSpec((1, QBLK, 3), lambda i, j: (i, j, 0)),
            pl.BlockSpec((1, 3, m), lambda i, j: (i, 0, 0)),
        ],
        out_specs=[
            pl.BlockSpec((1, QBLK, 3), lambda i, j: (i, j, 0)),
            pl.BlockSpec((1, QBLK, 3), lambda i, j: (i, j, 0)),
        ],
        out_shape=[
            jax.ShapeDtypeStruct((b, n, 3), jnp.float32),
            jax.ShapeDtypeStruct((b, n, 3), jnp.int32),
        ],
    )(unknown, kt)
    return dist, idx


# speedup vs baseline: 1.7750x; 1.7750x over previous
import functools

import jax
import jax.numpy as jnp
from jax.experimental import pallas as pl

QBLK = 512


def _threenn_block(u_ref, kt_ref, dist_ref, idx_ref):
    u = u_ref[0]
    kt = kt_ref[0]
    ux, uy, uz = u[:, 0:1], u[:, 1:2], u[:, 2:3]
    kx, ky, kz = kt[0:1, :], kt[1:2, :], kt[2:3, :]

    dot = jnp.dot(u.astype(jnp.bfloat16), kt.astype(jnp.bfloat16),
                  preferred_element_type=jnp.float32)
    su = ux * ux + uy * uy + uz * uz
    sk = kx * kx + ky * ky + kz * kz
    d = -2.0 * dot + su + sk

    iota = jax.lax.broadcasted_iota(jnp.int32, d.shape, 1)
    for k in range(3):
        mn = jnp.min(d, axis=1, keepdims=True)
        im = jnp.min(jnp.where(d == mn, iota, d.shape[-1]), axis=1, keepdims=True)
        dist_ref[0, :, pl.ds(k, 1)] = mn
        idx_ref[0, :, pl.ds(k, 1)] = im
        if k < 2:
            d = jnp.where(iota == im, jnp.inf, d)


@jax.jit
def kernel(unknown, known):
    b, n, _ = unknown.shape
    m = known.shape[1]
    kt = known.transpose(0, 2, 1)
    grid = (b, n // QBLK)
    dist, idx = pl.pallas_call(
        _threenn_block,
        grid=grid,
        in_specs=[
            pl.BlockSpec((1, QBLK, 3), lambda i, j: (i, j, 0)),
            pl.BlockSpec((1, 3, m), lambda i, j: (i, 0, 0)),
        ],
        out_specs=[
            pl.BlockSpec((1, QBLK, 3), lambda i, j: (i, j, 0)),
            pl.BlockSpec((1, QBLK, 3), lambda i, j: (i, j, 0)),
        ],
        out_shape=[
            jax.ShapeDtypeStruct((b, n, 3), jnp.float32),
            jax.ShapeDtypeStruct((b, n, 3), jnp.int32),
        ],
    )(unknown, kt)
    return dist, idx


# speedup vs baseline: 2.1634x; 1.2188x over previous
import functools

import jax
import jax.numpy as jnp
from jax.experimental import pallas as pl

QBLK = 512


def _threenn_block(u_ref, kt_ref, dist_ref, idx_ref):
    u = u_ref[0]
    kt = kt_ref[0]
    ux, uy, uz = u[:, 0:1], u[:, 1:2], u[:, 2:3]
    kx, ky, kz = kt[0:1, :], kt[1:2, :], kt[2:3, :]

    dotm2 = jnp.dot((-2.0 * u).astype(jnp.bfloat16), kt.astype(jnp.bfloat16),
                    preferred_element_type=jnp.float32)
    su = ux * ux + uy * uy + uz * uz
    sk = kx * kx + ky * ky + kz * kz
    d = dotm2 + su + sk

    iota = jax.lax.broadcasted_iota(jnp.int32, d.shape, 1).astype(jnp.float32)
    big = jnp.float32(d.shape[-1])
    for k in range(3):
        mn = jnp.min(d, axis=1, keepdims=True)
        im = jnp.min(jnp.where(d == mn, iota, big), axis=1, keepdims=True)
        dist_ref[0, :, pl.ds(k, 1)] = mn
        idx_ref[0, :, pl.ds(k, 1)] = im.astype(jnp.int32)
        if k < 2:
            d = jnp.where(iota == im, jnp.inf, d)


@jax.jit
def kernel(unknown, known):
    b, n, _ = unknown.shape
    m = known.shape[1]
    kt = known.transpose(0, 2, 1)
    grid = (b, n // QBLK)
    dist, idx = pl.pallas_call(
        _threenn_block,
        grid=grid,
        in_specs=[
            pl.BlockSpec((1, QBLK, 3), lambda i, j: (i, j, 0)),
            pl.BlockSpec((1, 3, m), lambda i, j: (i, 0, 0)),
        ],
        out_specs=[
            pl.BlockSpec((1, QBLK, 3), lambda i, j: (i, j, 0)),
            pl.BlockSpec((1, QBLK, 3), lambda i, j: (i, j, 0)),
        ],
        out_shape=[
            jax.ShapeDtypeStruct((b, n, 3), jnp.float32),
            jax.ShapeDtypeStruct((b, n, 3), jnp.int32),
        ],
    )(unknown, kt)
    return dist, idx


# speedup vs baseline: 2.2657x; 1.0473x over previous
import functools

import jax
import jax.numpy as jnp
from jax.experimental import pallas as pl

QBLK = 1024


def _threenn_block(u_ref, kt_ref, dist_ref, idx_ref):
    u = u_ref[0]
    kt = kt_ref[0]
    ux, uy, uz = u[:, 0:1], u[:, 1:2], u[:, 2:3]
    kx, ky, kz = kt[0:1, :], kt[1:2, :], kt[2:3, :]

    dotm2 = jnp.dot((-2.0 * u).astype(jnp.bfloat16), kt.astype(jnp.bfloat16),
                    preferred_element_type=jnp.float32)
    su = ux * ux + uy * uy + uz * uz
    sk = kx * kx + ky * ky + kz * kz
    d = dotm2 + su + sk

    iota = jax.lax.broadcasted_iota(jnp.int32, d.shape, 1).astype(jnp.float32)
    big = jnp.float32(d.shape[-1])
    for k in range(3):
        mn = jnp.min(d, axis=1, keepdims=True)
        im = jnp.min(jnp.where(d == mn, iota, big), axis=1, keepdims=True)
        dist_ref[0, :, pl.ds(k, 1)] = mn
        idx_ref[0, :, pl.ds(k, 1)] = im.astype(jnp.int32)
        if k < 2:
            d = jnp.where(iota == im, jnp.inf, d)


@jax.jit
def kernel(unknown, known):
    b, n, _ = unknown.shape
    m = known.shape[1]
    kt = known.transpose(0, 2, 1)
    grid = (b, n // QBLK)
    dist, idx = pl.pallas_call(
        _threenn_block,
        grid=grid,
        in_specs=[
            pl.BlockSpec((1, QBLK, 3), lambda i, j: (i, j, 0)),
            pl.BlockSpec((1, 3, m), lambda i, j: (i, 0, 0)),
        ],
        out_specs=[
            pl.BlockSpec((1, QBLK, 3), lambda i, j: (i, j, 0)),
            pl.BlockSpec((1, QBLK, 3), lambda i, j: (i, j, 0)),
        ],
        out_shape=[
            jax.ShapeDtypeStruct((b, n, 3), jnp.float32),
            jax.ShapeDtypeStruct((b, n, 3), jnp.int32),
        ],
    )(unknown, kt)
    return dist, idx


# speedup vs baseline: 2.2659x; 1.0001x over previous
import functools

import jax
import jax.numpy as jnp
from jax.experimental import pallas as pl

QBLK = 2048


def _threenn_block(u_ref, kt_ref, dist_ref, idx_ref):
    u = u_ref[0]
    kt = kt_ref[0]
    ux, uy, uz = u[:, 0:1], u[:, 1:2], u[:, 2:3]
    kx, ky, kz = kt[0:1, :], kt[1:2, :], kt[2:3, :]

    dotm2 = jnp.dot((-2.0 * u).astype(jnp.bfloat16), kt.astype(jnp.bfloat16),
                    preferred_element_type=jnp.float32)
    su = ux * ux + uy * uy + uz * uz
    sk = kx * kx + ky * ky + kz * kz
    d = dotm2 + su + sk

    iota = jax.lax.broadcasted_iota(jnp.int32, d.shape, 1).astype(jnp.float32)
    big = jnp.float32(d.shape[-1])
    for k in range(3):
        mn = jnp.min(d, axis=1, keepdims=True)
        im = jnp.min(jnp.where(d == mn, iota, big), axis=1, keepdims=True)
        dist_ref[0, :, pl.ds(k, 1)] = mn
        idx_ref[0, :, pl.ds(k, 1)] = im.astype(jnp.int32)
        if k < 2:
            d = jnp.where(iota == im, jnp.inf, d)


@jax.jit
def kernel(unknown, known):
    b, n, _ = unknown.shape
    m = known.shape[1]
    kt = known.transpose(0, 2, 1)
    grid = (b, n // QBLK)
    dist, idx = pl.pallas_call(
        _threenn_block,
        grid=grid,
        in_specs=[
            pl.BlockSpec((1, QBLK, 3), lambda i, j: (i, j, 0)),
            pl.BlockSpec((1, 3, m), lambda i, j: (i, 0, 0)),
        ],
        out_specs=[
            pl.BlockSpec((1, QBLK, 3), lambda i, j: (i, j, 0)),
            pl.BlockSpec((1, QBLK, 3), lambda i, j: (i, j, 0)),
        ],
        out_shape=[
            jax.ShapeDtypeStruct((b, n, 3), jnp.float32),
            jax.ShapeDtypeStruct((b, n, 3), jnp.int32),
        ],
    )(unknown, kt)
    return dist, idx
